# baseline (speedup 1.0000x reference)
import functools

import jax
import jax.numpy as jnp
from jax import lax
from jax.experimental import pallas as pl
from jax.experimental.pallas import tpu as pltpu
from jax.experimental.pallas import tpu_sc as plsc

_NUM_TOKENS = 16384
_NUM_EXPERTS = 64
_LANES = 16
_NUM_WORKERS = 32
_ROWS_PER_W = _NUM_TOKENS // _NUM_WORKERS
_GROUP = 16
_VPT = _NUM_EXPERTS // _LANES
_CHUNK_T = 128
_NCHUNK = _ROWS_PER_W // _CHUNK_T
_CHUNK_W = _CHUNK_T * _NUM_EXPERTS

_mesh = plsc.VectorSubcoreMesh(
    core_axis_name="c", subcore_axis_name="s", num_cores=2, num_subcores=16
)


@functools.partial(
    pl.kernel,
    out_type=jax.ShapeDtypeStruct((_NUM_TOKENS * 2,), jnp.float32),
    mesh=_mesh,
    scratch_types=[
        pltpu.VMEM((2, _CHUNK_W), jnp.float32),
        pltpu.VMEM((_NUM_EXPERTS,), jnp.float32),
        pltpu.VMEM((_NUM_EXPERTS,), jnp.float32),
        pltpu.VMEM((_ROWS_PER_W * 2,), jnp.float32),
        pltpu.SemaphoreType.DMA,
        pltpu.SemaphoreType.DMA,
    ],
    compiler_params=pltpu.CompilerParams(
        needs_layout_passes=False, use_tc_tiling_on_sc=False
    ),
)
def _moe_sc(rw_hbm, wops_hbm, wimms_hbm, out_hbm, rw_v, wops_v, wimms_v, out_v,
            sem0, sem1):
    wid = lax.axis_index("s") * 2 + lax.axis_index("c")
    in_base = wid * _ROWS_PER_W * _NUM_EXPERTS
    out_base = wid * _ROWS_PER_W * 2
    sems = [sem0, sem1]

    pltpu.sync_copy(wops_hbm, wops_v)
    pltpu.sync_copy(wimms_hbm, wimms_v)

    copies = [
        pltpu.async_copy(
            rw_hbm.at[pl.ds(in_base + ch * _CHUNK_W, _CHUNK_W)],
            rw_v.at[ch % 2],
            sems[ch % 2],
        )
        for ch in range(2)
    ]

    wops = [wops_v[pl.ds(j * _LANES, _LANES)] for j in range(_VPT)]
    wimms = [wimms_v[pl.ds(j * _LANES, _LANES)] for j in range(_VPT)]
    lane = lax.iota(jnp.int32, _LANES)
    m15 = lane == (_LANES - 1)

    for ch in range(_NCHUNK):
        copies[ch].wait()
        buf = rw_v.at[ch % 2]
        t_off = ch * _CHUNK_T

        def group(g, carry):
            t0 = g * _GROUP
            for k in range(_GROUP):
                t = t0 + k
                row = [buf[pl.ds(t * _NUM_EXPERTS + j * _LANES, _LANES)]
                       for j in range(_VPT)]
                c0 = row[0] * wops[0]
                c1 = row[0] * wimms[0]
                for j in range(1, _VPT):
                    c0 = c0 + row[j] * wops[j]
                    c1 = c1 + row[j] * wimms[j]
                s0 = plsc.cumsum(c0)
                s1 = plsc.cumsum(c1)
                i0 = jnp.broadcast_to((t_off + t) * 2, (_LANES,))
                plsc.store_scatter(out_v, [i0], s0, mask=m15)
                plsc.store_scatter(out_v, [i0 + 1], s1, mask=m15)
            return carry

        lax.fori_loop(0, _CHUNK_T // _GROUP, group, 0)
        if ch + 2 < _NCHUNK:
            copies.append(
                pltpu.async_copy(
                    rw_hbm.at[pl.ds(in_base + (ch + 2) * _CHUNK_W, _CHUNK_W)],
                    rw_v.at[ch % 2],
                    sems[ch % 2],
                )
            )

    pltpu.sync_copy(out_v, out_hbm.at[pl.ds(out_base, _ROWS_PER_W * 2)])


def kernel(router_weights, W_values):
    w_ops = jnp.asarray(W_values[:, 0], jnp.float32)
    w_imms = jnp.asarray(W_values[:, 1], jnp.float32)
    rw_flat = router_weights.reshape(_NUM_TOKENS * _NUM_EXPERTS)
    out = _moe_sc(rw_flat, w_ops, w_imms)
    return out.reshape(_NUM_TOKENS, 2)
